# baseline (speedup 1.0000x reference)
import functools

import jax
import jax.numpy as jnp
from jax import lax
from jax.experimental import pallas as pl
from jax.experimental.pallas import tpu as pltpu
from jax.experimental.pallas import tpu_sc as plsc

B, L, D, E = 16384, 200, 10, 16
N = B * L

_mesh = plsc.VectorSubcoreMesh(core_axis_name="c", subcore_axis_name="s")


@functools.partial(
    pl.kernel,
    mesh=_mesh,
    out_type=jax.ShapeDtypeStruct((L, E, B), jnp.float32),
    scratch_types=[
        pltpu.VMEM((16,), jnp.float32),
    ],
    compiler_params=pltpu.CompilerParams(needs_layout_passes=False),
)
def _encode(x_hbm, emb_hbm, a_hbm, out_hbm, xb):
    xb[...] = jnp.zeros((16,), jnp.float32)


def kernel(number, emb, prelu_a):
    x = number.reshape(N * D)
    a16 = jnp.broadcast_to(prelu_a.astype(jnp.float32), (16,))
    o = _encode(x, emb, a16)
    return jnp.transpose(o, (2, 0, 1))


# speedup vs baseline: 11.9770x; 11.9770x over previous
import functools

import jax
import jax.numpy as jnp
from jax import lax
from jax.experimental import pallas as pl
from jax.experimental.pallas import tpu as pltpu
from jax.experimental.pallas import tpu_sc as plsc

B, L, D, E = 16384, 200, 10, 16
NW = 32
BPW = B // NW
BW = 256
LC = 8
NBH = BPW // BW
NLB = L // LC
NBLK = NBH * NLB

_mesh = plsc.VectorSubcoreMesh(core_axis_name="c", subcore_axis_name="s")


@functools.partial(
    pl.kernel,
    mesh=_mesh,
    out_type=jax.ShapeDtypeStruct((L, E, B), jnp.float32),
    scratch_types=[
        pltpu.VMEM((D, LC, BW), jnp.float32),
        pltpu.VMEM((D, LC, BW), jnp.float32),
        pltpu.VMEM((LC, E, BW), jnp.float32),
        pltpu.VMEM((LC, E, BW), jnp.float32),
        pltpu.VMEM((D * E,), jnp.float32),
        pltpu.VMEM((16,), jnp.float32),
        pltpu.SemaphoreType.DMA,
        pltpu.SemaphoreType.DMA,
        pltpu.SemaphoreType.DMA,
        pltpu.SemaphoreType.DMA,
    ],
    compiler_params=pltpu.CompilerParams(needs_layout_passes=False),
)
def _encode(x_hbm, emb_hbm, a_hbm, out_hbm,
            xin0, xin1, yo0, yo1, tb, ab, xs0, xs1, ys0, ys1):
    wid = lax.axis_index("s") * 2 + lax.axis_index("c")
    tile_b0 = wid * BPW

    xins = (xin0, xin1)
    youts = (yo0, yo1)
    xsems = (xs0, xs1)
    ysems = (ys0, ys1)

    pltpu.sync_copy(emb_hbm, tb)
    pltpu.sync_copy(a_hbm, ab)
    a = ab[...]
    for i in range(D * E // 16):
        v = tb[pl.ds(16 * i, 16)]
        tb[pl.ds(16 * i, 16)] = jnp.maximum(v, 0.0) + a * jnp.minimum(v, 0.0)

    def block_coords(m):
        bh = m // NLB
        l0 = (m % NLB) * LC
        return tile_b0 + bh * BW, l0

    def in_copy(m, buf):
        b0, l0 = block_coords(m)
        return pltpu.make_async_copy(
            x_hbm.at[:, pl.ds(l0, LC), pl.ds(b0, BW)], xins[buf], xsems[buf])

    def out_copy(m, buf):
        b0, l0 = block_coords(m)
        return pltpu.make_async_copy(
            youts[buf], out_hbm.at[pl.ds(l0, LC), :, pl.ds(b0, BW)], ysems[buf])

    def compute_block(buf):
        xin = xins[buf]
        yo = youts[buf]

        @plsc.parallel_loop(0, LC * (BW // 16), unroll=2)
        def unit(u):
            l = u >> 4
            bs = (u & 15) * 16
            maxv = xin[0, l, pl.ds(bs, 16)]
            maxi = jnp.zeros((16,), jnp.int32)
            for d in range(1, D):
                xd = xin[d, l, pl.ds(bs, 16)]
                m = xd > maxv
                maxv = jnp.where(m, xd, maxv)
                maxi = jnp.where(m, jnp.int32(d), maxi)
            for e in range(E):
                yo[l, e, pl.ds(bs, 16)] = plsc.load_gather(
                    tb, [maxi + jnp.int32(e * D)])

    in_copy(0, 0).start()
    in_copy(1, 1).start()

    def pair_body(g, carry):
        for buf in range(2):
            m = g * 2 + buf
            in_copy(m, buf).wait()

            @pl.when(m >= 2)
            def _():
                out_copy(m - 2, buf).wait()

            compute_block(buf)
            out_copy(m, buf).start()

            @pl.when(m + 2 < NBLK)
            def _():
                in_copy(m + 2, buf).start()

        return carry

    lax.fori_loop(0, NBLK // 2, pair_body, 0)
    out_copy(NBLK - 2, 0).wait()
    out_copy(NBLK - 1, 1).wait()


def kernel(number, emb, prelu_a):
    x_t = jnp.transpose(number, (2, 1, 0))
    emb_flat = jnp.transpose(emb).reshape(D * E)
    a16 = jnp.broadcast_to(prelu_a.astype(jnp.float32), (16,))
    o = _encode(x_t, emb_flat, a16)
    return jnp.transpose(o, (2, 0, 1))
